# baseline (speedup 1.0000x reference)
import jax
import jax.numpy as jnp
from jax.experimental import pallas as pl

_NUM_CLASSES = 1000
_ROWS = 16384
_BLOCK_ROWS = 1024


def _one_hot_block(x_ref, o_ref):
    idx = x_ref[:, 0]
    iota = jax.lax.broadcasted_iota(jnp.int32, (_BLOCK_ROWS, _NUM_CLASSES), 1)
    o_ref[...] = (idx[:, None] == iota).astype(jnp.int32)


def kernel(x):
    idx = x.astype(jnp.int32)
    return pl.pallas_call(
        _one_hot_block,
        grid=(_ROWS // _BLOCK_ROWS,),
        in_specs=[pl.BlockSpec((_BLOCK_ROWS, 1), lambda i: (i, 0))],
        out_specs=pl.BlockSpec((_BLOCK_ROWS, _NUM_CLASSES), lambda i: (i, 0)),
        out_shape=jax.ShapeDtypeStruct((_ROWS, _NUM_CLASSES), jnp.int32),
    )(idx)


# speedup vs baseline: 1.0075x; 1.0075x over previous
import jax
import jax.numpy as jnp
from jax.experimental import pallas as pl
from jax.experimental.pallas import tpu as pltpu

_NUM_CLASSES = 1000
_ROWS = 16384
_BLOCK_ROWS = 1024


def _one_hot_block(x_ref, o_ref):
    idx = x_ref[:, 0]
    iota = jax.lax.broadcasted_iota(jnp.int32, (_BLOCK_ROWS, _NUM_CLASSES), 1)
    o_ref[...] = (idx[:, None] == iota).astype(jnp.int32)


def kernel(x):
    idx = x.astype(jnp.int32)
    return pl.pallas_call(
        _one_hot_block,
        grid=(_ROWS // _BLOCK_ROWS,),
        in_specs=[pl.BlockSpec((_BLOCK_ROWS, 1), lambda i: (i, 0))],
        out_specs=pl.BlockSpec((_BLOCK_ROWS, _NUM_CLASSES), lambda i: (i, 0)),
        out_shape=jax.ShapeDtypeStruct((_ROWS, _NUM_CLASSES), jnp.int32),
        compiler_params=pltpu.CompilerParams(
            dimension_semantics=("parallel",)),
    )(idx)
